# baseline (speedup 1.0000x reference)
import functools

import jax
import jax.numpy as jnp
from jax import lax
from jax.experimental import pallas as pl
from jax.experimental.pallas import tpu as pltpu
from jax.experimental.pallas import tpu_sc as plsc

N = 10000
E = 320000
D = 128
N_PAD = 10240
CHUNK = 80
NBUF = 4
MM_BLOCK = 1024


def _mm_kernel(nf_ref, w1_ref, w2_ref, a_ref, b_ref):
    x = nf_ref[...]
    a_ref[...] = jnp.dot(x, w1_ref[...], preferred_element_type=jnp.float32)
    b_ref[...] = jnp.dot(x, w2_ref[...], preferred_element_type=jnp.float32)


def _precompute_tables(nf_pad, w1, w2):
    grid = N_PAD // MM_BLOCK
    out_sds = jax.ShapeDtypeStruct((N_PAD, D), jnp.float32)
    return pl.pallas_call(
        _mm_kernel,
        grid=(grid,),
        in_specs=[
            pl.BlockSpec((MM_BLOCK, D), lambda i: (i, 0)),
            pl.BlockSpec((D, D), lambda i: (0, 0)),
            pl.BlockSpec((D, D), lambda i: (0, 0)),
        ],
        out_specs=[
            pl.BlockSpec((MM_BLOCK, D), lambda i: (i, 0)),
            pl.BlockSpec((MM_BLOCK, D), lambda i: (i, 0)),
        ],
        out_shape=[out_sds, out_sds],
    )(nf_pad, w1, w2)


def _make_edge_kernel(num_workers, epw):
    mesh = plsc.VectorSubcoreMesh(core_axis_name="c", subcore_axis_name="s")
    n_chunks = epw // CHUNK

    rows_t = pltpu.VMEM((CHUNK, D), jnp.float32)

    @functools.partial(
        pl.kernel,
        mesh=mesh,
        out_type=jax.ShapeDtypeStruct((E, D), jnp.float32),
        scratch_types=(
            [pltpu.VMEM((epw,), jnp.int32)] * 3
            + [rows_t] * NBUF + [rows_t] * NBUF
            + [pltpu.SemaphoreType.DMA] * (3 * NBUF)
        ),
    )
    def edge_kernel(a_hbm, b_hbm, src_hbm, dst_hbm, conn_hbm, out_hbm,
                    *refs):
        src_v, dst_v, conn_v = refs[0:3]
        rows_a = refs[3:3 + NBUF]
        rows_b = refs[3 + NBUF:3 + 2 * NBUF]
        sems = refs[3 + 2 * NBUF:]
        sem_a = sems[0:NBUF]
        sem_b = sems[NBUF:2 * NBUF]
        sem_w = sems[2 * NBUF:3 * NBUF]

        nc = num_workers // 16
        wid = lax.axis_index("s") * nc + lax.axis_index("c")
        base = wid * epw
        pltpu.sync_copy(src_hbm.at[pl.ds(base, epw)], src_v)
        pltpu.sync_copy(dst_hbm.at[pl.ds(base, epw)], dst_v)
        pltpu.sync_copy(conn_hbm.at[pl.ds(base, epw)], conn_v)

        def mask_body(j, carry):
            sl = pl.ds(j * 16, 16)
            s = src_v[sl]
            dd = dst_v[sl]
            m = (conn_v[sl] == 1) & (s < dd)
            src_v[sl] = jnp.where(m, s, N)
            dst_v[sl] = jnp.where(m, dd, N)
            return carry

        lax.fori_loop(0, epw // 16, mask_body, 0, unroll=4)

        def issue(k, b):
            off = k * CHUNK
            pltpu.async_copy(a_hbm.at[src_v.at[pl.ds(off, CHUNK)]],
                             rows_a[b], sem_a[b])
            pltpu.async_copy(b_hbm.at[dst_v.at[pl.ds(off, CHUNK)]],
                             rows_b[b], sem_b[b])

        def process(k, b):
            off = k * CHUNK
            pltpu.make_async_copy(a_hbm.at[pl.ds(0, CHUNK)],
                                  rows_a[b], sem_a[b]).wait()
            pltpu.make_async_copy(b_hbm.at[pl.ds(0, CHUNK)],
                                  rows_b[b], sem_b[b]).wait()

            @pl.when(k >= NBUF)
            def _():
                pltpu.make_async_copy(rows_a[b], out_hbm.at[pl.ds(0, CHUNK)],
                                      sem_w[b]).wait()

            pltpu.async_copy(rows_a[b], out_hbm.at[pl.ds(base + off, CHUNK)],
                             sem_w[b])

        for b in range(NBUF):
            issue(b, b)

        def outer_body(i, carry):
            for b in range(NBUF):
                k = NBUF * i + b
                process(k, b)

                @pl.when(k + NBUF < n_chunks)
                def _():
                    issue(k + NBUF, b)
            return carry

        lax.fori_loop(0, n_chunks // NBUF, outer_body, 0)
        for k in range(n_chunks - n_chunks % NBUF, n_chunks):
            b = k % NBUF
            process(k, b)

        for b in range(NBUF):
            pltpu.make_async_copy(rows_a[b], out_hbm.at[pl.ds(0, CHUNK)],
                                  sem_w[b]).wait()

    return edge_kernel


def kernel(nf, edge_index, connected, W):
    ei = edge_index.astype(jnp.int32)
    src = ei[0]
    dst = ei[1]
    conn = connected.astype(jnp.int32)
    nf_pad = jnp.zeros((N_PAD, D), jnp.float32).at[:N, :].set(nf)
    w1 = W[:D, :]
    w2 = W[D:, :]
    a, b = _precompute_tables(nf_pad, w1, w2)

    info = plsc.get_sparse_core_info()
    num_workers = info.num_cores * info.num_subcores
    epw = E // num_workers
    edge_fn = _make_edge_kernel(num_workers, epw)
    return edge_fn(a, b, src, dst, conn)


# speedup vs baseline: 11.7848x; 11.7848x over previous
import jax
import jax.numpy as jnp
from jax import lax
from jax.experimental import pallas as pl
from jax.experimental.pallas import tpu as pltpu

N = 10000
E = 320000
D = 128
N_PAD = 10240
BLK = 2560
MM_BLOCK = 1024


def _mm_kernel(nf_ref, w1_ref, w2_ref, a_ref, b_ref):
    x = nf_ref[...]
    a_ref[...] = jnp.dot(x, w1_ref[...], preferred_element_type=jnp.float32)
    b_ref[...] = jnp.dot(x, w2_ref[...], preferred_element_type=jnp.float32)


def _precompute_tables(nf_pad, w1, w2):
    grid = N_PAD // MM_BLOCK
    out_sds = jax.ShapeDtypeStruct((N_PAD, D), jnp.float32)
    return pl.pallas_call(
        _mm_kernel,
        grid=(grid,),
        in_specs=[
            pl.BlockSpec((MM_BLOCK, D), lambda i: (i, 0)),
            pl.BlockSpec((D, D), lambda i: (0, 0)),
            pl.BlockSpec((D, D), lambda i: (0, 0)),
        ],
        out_specs=[
            pl.BlockSpec((MM_BLOCK, D), lambda i: (i, 0)),
            pl.BlockSpec((MM_BLOCK, D), lambda i: (i, 0)),
        ],
        out_shape=[out_sds, out_sds],
    )(nf_pad, w1, w2)


def _mask_kernel(src_ref, dst_ref, conn_ref, ms_ref, md_ref):
    s = src_ref[...]
    d = dst_ref[...]
    m = (conn_ref[...] == 1) & (s < d)
    ms_ref[...] = jnp.where(m, s, N)
    md_ref[...] = jnp.where(m, d, N)


def _fold_mask(src3, dst3, conn3, nblk):
    out_sds = jax.ShapeDtypeStruct((nblk, 1, BLK), jnp.int32)
    spec = pl.BlockSpec((1, 1, BLK), lambda i: (i, 0, 0))
    return pl.pallas_call(
        _mask_kernel,
        grid=(nblk,),
        in_specs=[spec, spec, spec],
        out_specs=[spec, spec],
        out_shape=[out_sds, out_sds],
    )(src3, dst3, conn3)


def _edge_kernel(ms_ref, md_ref, a_ref, b_ref, out_ref):
    def row(i, carry):
        ia = ms_ref[0, 0, i]
        ib = md_ref[0, 0, i]
        v = a_ref[pl.ds(ia, 1), :] + b_ref[pl.ds(ib, 1), :]
        out_ref[pl.ds(i, 1), :] = jnp.maximum(v, v * 0.01)
        return carry

    lax.fori_loop(0, BLK, row, 0, unroll=8)


def _gather_compute(ms3, md3, a, b, nblk):
    idx_spec = pl.BlockSpec((1, 1, BLK), lambda i: (i, 0, 0),
                            memory_space=pltpu.SMEM)
    tab_spec = pl.BlockSpec((N_PAD, D), lambda i: (0, 0))
    return pl.pallas_call(
        _edge_kernel,
        grid=(nblk,),
        in_specs=[idx_spec, idx_spec, tab_spec, tab_spec],
        out_specs=pl.BlockSpec((BLK, D), lambda i: (i, 0)),
        out_shape=jax.ShapeDtypeStruct((E, D), jnp.float32),
    )(ms3, md3, a, b)


def kernel(nf, edge_index, connected, W):
    ei = edge_index.astype(jnp.int32)
    nblk = E // BLK
    src3 = ei[0].reshape(nblk, 1, BLK)
    dst3 = ei[1].reshape(nblk, 1, BLK)
    conn3 = connected.astype(jnp.int32).reshape(nblk, 1, BLK)
    nf_pad = jnp.zeros((N_PAD, D), jnp.float32).at[:N, :].set(nf)
    w1 = W[:D, :]
    w2 = W[D:, :]
    a, b = _precompute_tables(nf_pad, w1, w2)
    ms3, md3 = _fold_mask(src3, dst3, conn3, nblk)
    return _gather_compute(ms3, md3, a, b, nblk)


# speedup vs baseline: 13.1502x; 1.1159x over previous
import jax
import jax.numpy as jnp
from jax import lax
from jax.experimental import pallas as pl
from jax.experimental.pallas import tpu as pltpu

N = 10000
E = 320000
D = 128
N_PAD = 10240
BLK = 2560
MM_BLOCK = 1024


def _mm_kernel(nf_ref, w1_ref, w2_ref, a_ref, b_ref):
    x = nf_ref[...]
    a_ref[...] = jnp.dot(x, w1_ref[...], preferred_element_type=jnp.float32)
    b_ref[...] = jnp.dot(x, w2_ref[...], preferred_element_type=jnp.float32)


def _precompute_tables(nf_pad, w1, w2):
    grid = N_PAD // MM_BLOCK
    out_sds = jax.ShapeDtypeStruct((N_PAD, D), jnp.float32)
    return pl.pallas_call(
        _mm_kernel,
        grid=(grid,),
        in_specs=[
            pl.BlockSpec((MM_BLOCK, D), lambda i: (i, 0)),
            pl.BlockSpec((D, D), lambda i: (0, 0)),
            pl.BlockSpec((D, D), lambda i: (0, 0)),
        ],
        out_specs=[
            pl.BlockSpec((MM_BLOCK, D), lambda i: (i, 0)),
            pl.BlockSpec((MM_BLOCK, D), lambda i: (i, 0)),
        ],
        out_shape=[out_sds, out_sds],
    )(nf_pad, w1, w2)


def _mask_kernel(src_ref, dst_ref, conn_ref, mp_ref):
    s = src_ref[...]
    d = dst_ref[...]
    m = (conn_ref[...] == 1) & (s < d)
    mp_ref[...] = jnp.where(m, s | (d << 14), N | (N << 14))


def _fold_mask(src3, dst3, conn3, nblk):
    out_sds = jax.ShapeDtypeStruct((nblk, 1, BLK), jnp.int32)
    spec = pl.BlockSpec((1, 1, BLK), lambda i: (i, 0, 0))
    return pl.pallas_call(
        _mask_kernel,
        grid=(nblk,),
        in_specs=[spec, spec, spec],
        out_specs=spec,
        out_shape=out_sds,
    )(src3, dst3, conn3)


def _edge_kernel(mp_ref, a_ref, b_ref, out_ref):
    def row(i, carry):
        vv = mp_ref[0, 0, i]
        ia = vv & 16383
        ib = lax.shift_right_logical(vv, 14)
        v = a_ref[pl.ds(ia, 1), :] + b_ref[pl.ds(ib, 1), :]
        out_ref[pl.ds(i, 1), :] = jnp.maximum(v, v * 0.01)
        return carry

    lax.fori_loop(0, BLK, row, 0, unroll=16)


def _gather_compute(mp3, a, b, nblk):
    idx_spec = pl.BlockSpec((1, 1, BLK), lambda i: (i, 0, 0),
                            memory_space=pltpu.SMEM)
    tab_spec = pl.BlockSpec((N_PAD, D), lambda i: (0, 0))
    return pl.pallas_call(
        _edge_kernel,
        grid=(nblk,),
        in_specs=[idx_spec, tab_spec, tab_spec],
        out_specs=pl.BlockSpec((BLK, D), lambda i: (i, 0)),
        out_shape=jax.ShapeDtypeStruct((E, D), jnp.float32),
    )(mp3, a, b)


def kernel(nf, edge_index, connected, W):
    ei = edge_index.astype(jnp.int32)
    nblk = E // BLK
    src3 = ei[0].reshape(nblk, 1, BLK)
    dst3 = ei[1].reshape(nblk, 1, BLK)
    conn3 = connected.astype(jnp.int32).reshape(nblk, 1, BLK)
    nf_pad = jnp.zeros((N_PAD, D), jnp.float32).at[:N, :].set(nf)
    w1 = W[:D, :]
    w2 = W[D:, :]
    a, b = _precompute_tables(nf_pad, w1, w2)
    mp3 = _fold_mask(src3, dst3, conn3, nblk)
    return _gather_compute(mp3, a, b, nblk)


# speedup vs baseline: 13.5430x; 1.0299x over previous
import jax
import jax.numpy as jnp
from jax import lax
from jax.experimental import pallas as pl
from jax.experimental.pallas import tpu as pltpu

N = 10000
E = 320000
D = 128
N_PAD = 10240
BLK = 2560
MM_BLOCK = 1024


def _mm_kernel(nf_ref, w1_ref, w2_ref, a_ref, b_ref):
    x = nf_ref[...]
    a_ref[...] = jnp.dot(x, w1_ref[...], preferred_element_type=jnp.float32)
    b_ref[...] = jnp.dot(x, w2_ref[...], preferred_element_type=jnp.float32)


def _precompute_tables(nf_pad, w1, w2):
    grid = N_PAD // MM_BLOCK
    out_sds = jax.ShapeDtypeStruct((N_PAD, D), jnp.float32)
    return pl.pallas_call(
        _mm_kernel,
        grid=(grid,),
        in_specs=[
            pl.BlockSpec((MM_BLOCK, D), lambda i: (i, 0)),
            pl.BlockSpec((D, D), lambda i: (0, 0)),
            pl.BlockSpec((D, D), lambda i: (0, 0)),
        ],
        out_specs=[
            pl.BlockSpec((MM_BLOCK, D), lambda i: (i, 0)),
            pl.BlockSpec((MM_BLOCK, D), lambda i: (i, 0)),
        ],
        out_shape=[out_sds, out_sds],
    )(nf_pad, w1, w2)


def _mask_kernel(src_ref, dst_ref, conn_ref, mp_ref):
    s = src_ref[...]
    d = dst_ref[...]
    m = (conn_ref[...] == 1) & (s < d)
    mp_ref[...] = jnp.where(m, s | (d << 14), N | (N << 14))


def _fold_mask(src3, dst3, conn3, nblk):
    out_sds = jax.ShapeDtypeStruct((nblk, 1, BLK), jnp.int32)
    spec = pl.BlockSpec((1, 1, BLK), lambda i: (i, 0, 0))
    return pl.pallas_call(
        _mask_kernel,
        grid=(nblk,),
        in_specs=[spec, spec, spec],
        out_specs=spec,
        out_shape=out_sds,
    )(src3, dst3, conn3)


def _edge_kernel(mp_ref, a_ref, b_ref, out_ref):
    def row(i, carry):
        vv = mp_ref[0, 0, i]
        ia = vv & 16383
        ib = lax.shift_right_logical(vv, 14)
        v = a_ref[pl.ds(ia, 1), :] + b_ref[pl.ds(ib, 1), :]
        out_ref[pl.ds(i, 1), :] = jnp.maximum(v, v * 0.01)
        return carry

    lax.fori_loop(0, BLK, row, 0, unroll=32)


def _gather_compute(mp3, a, b, nblk):
    idx_spec = pl.BlockSpec((1, 1, BLK), lambda i: (i, 0, 0),
                            memory_space=pltpu.SMEM)
    tab_spec = pl.BlockSpec((N_PAD, D), lambda i: (0, 0))
    return pl.pallas_call(
        _edge_kernel,
        grid=(nblk,),
        in_specs=[idx_spec, tab_spec, tab_spec],
        out_specs=pl.BlockSpec((BLK, D), lambda i: (i, 0)),
        out_shape=jax.ShapeDtypeStruct((E, D), jnp.float32),
    )(mp3, a, b)


def kernel(nf, edge_index, connected, W):
    ei = edge_index.astype(jnp.int32)
    nblk = E // BLK
    src3 = ei[0].reshape(nblk, 1, BLK)
    dst3 = ei[1].reshape(nblk, 1, BLK)
    conn3 = connected.astype(jnp.int32).reshape(nblk, 1, BLK)
    nf_pad = jnp.zeros((N_PAD, D), jnp.float32).at[:N, :].set(nf)
    w1 = W[:D, :]
    w2 = W[D:, :]
    a, b = _precompute_tables(nf_pad, w1, w2)
    mp3 = _fold_mask(src3, dst3, conn3, nblk)
    return _gather_compute(mp3, a, b, nblk)


# speedup vs baseline: 14.2240x; 1.0503x over previous
import jax
import jax.numpy as jnp
from jax import lax
from jax.experimental import pallas as pl
from jax.experimental.pallas import tpu as pltpu

N = 10000
E = 320000
D = 128
N_PAD = 10240
BLK = 10000
MM_BLOCK = 1024


def _mm_kernel(nf_ref, w1_ref, w2_ref, a_ref, b_ref):
    x = nf_ref[...]
    a_ref[...] = jnp.dot(x, w1_ref[...], preferred_element_type=jnp.float32)
    b_ref[...] = jnp.dot(x, w2_ref[...], preferred_element_type=jnp.float32)


def _precompute_tables(nf_pad, w1, w2):
    grid = N_PAD // MM_BLOCK
    out_sds = jax.ShapeDtypeStruct((N_PAD, D), jnp.float32)
    return pl.pallas_call(
        _mm_kernel,
        grid=(grid,),
        in_specs=[
            pl.BlockSpec((MM_BLOCK, D), lambda i: (i, 0)),
            pl.BlockSpec((D, D), lambda i: (0, 0)),
            pl.BlockSpec((D, D), lambda i: (0, 0)),
        ],
        out_specs=[
            pl.BlockSpec((MM_BLOCK, D), lambda i: (i, 0)),
            pl.BlockSpec((MM_BLOCK, D), lambda i: (i, 0)),
        ],
        out_shape=[out_sds, out_sds],
    )(nf_pad, w1, w2)


def _mask_kernel(src_ref, dst_ref, conn_ref, mp_ref):
    s = src_ref[...]
    d = dst_ref[...]
    m = (conn_ref[...] == 1) & (s < d)
    mp_ref[...] = jnp.where(m, s | (d << 14), N | (N << 14))


def _fold_mask(src3, dst3, conn3, nblk):
    out_sds = jax.ShapeDtypeStruct((nblk, 1, BLK), jnp.int32)
    spec = pl.BlockSpec((1, 1, BLK), lambda i: (i, 0, 0))
    return pl.pallas_call(
        _mask_kernel,
        grid=(nblk,),
        in_specs=[spec, spec, spec],
        out_specs=spec,
        out_shape=out_sds,
    )(src3, dst3, conn3)


def _edge_kernel(mp_ref, a_ref, b_ref, out_ref):
    def row(i, carry):
        vv = mp_ref[0, 0, i]
        ia = vv & 16383
        ib = lax.shift_right_logical(vv, 14)
        v = a_ref[pl.ds(ia, 1), :] + b_ref[pl.ds(ib, 1), :]
        out_ref[pl.ds(i, 1), :] = jnp.maximum(v, v * 0.01)
        return carry

    lax.fori_loop(0, BLK, row, 0, unroll=32)


def _gather_compute(mp3, a, b, nblk):
    idx_spec = pl.BlockSpec((1, 1, BLK), lambda i: (i, 0, 0),
                            memory_space=pltpu.SMEM)
    tab_spec = pl.BlockSpec((N_PAD, D), lambda i: (0, 0))
    return pl.pallas_call(
        _edge_kernel,
        grid=(nblk,),
        in_specs=[idx_spec, tab_spec, tab_spec],
        out_specs=pl.BlockSpec((BLK, D), lambda i: (i, 0)),
        out_shape=jax.ShapeDtypeStruct((E, D), jnp.float32),
    )(mp3, a, b)


def kernel(nf, edge_index, connected, W):
    ei = edge_index.astype(jnp.int32)
    nblk = E // BLK
    src3 = ei[0].reshape(nblk, 1, BLK)
    dst3 = ei[1].reshape(nblk, 1, BLK)
    conn3 = connected.astype(jnp.int32).reshape(nblk, 1, BLK)
    nf_pad = jnp.zeros((N_PAD, D), jnp.float32).at[:N, :].set(nf)
    w1 = W[:D, :]
    w2 = W[D:, :]
    a, b = _precompute_tables(nf_pad, w1, w2)
    mp3 = _fold_mask(src3, dst3, conn3, nblk)
    return _gather_compute(mp3, a, b, nblk)
